# baseline (speedup 1.0000x reference)
import dataclasses
import functools

import jax
import jax.numpy as jnp
from jax import lax
from jax.experimental import pallas as pl
from jax.experimental.pallas import tpu as pltpu
from jax.experimental.pallas import tpu_sc as plsc

_NUM_CORES = 2
_NUM_SUBCORES = 16
_NW = _NUM_CORES * _NUM_SUBCORES
_LANES = 16
_BLOCK = 128


@functools.lru_cache(maxsize=None)
def _build_sc_lerp_gather(t_total: int, nc: int, d: int):
    assert t_total % (_NW * _BLOCK) == 0
    rows_per_w = t_total // _NW
    nblk = rows_per_w // _BLOCK
    assert nblk % 2 == 0
    mesh = plsc.VectorSubcoreMesh(core_axis_name="c", subcore_axis_name="s")
    cparams = pltpu.CompilerParams()
    if "needs_layout_passes" in pltpu.CompilerParams.__dataclass_fields__:
        cparams = dataclasses.replace(cparams, needs_layout_passes=False)

    @functools.partial(
        pl.kernel,
        out_type=jax.ShapeDtypeStruct((t_total, d), jnp.float32),
        mesh=mesh,
        compiler_params=cparams,
        scratch_types=[
            pltpu.VMEM((rows_per_w,), jnp.float32),
            pltpu.VMEM((rows_per_w,), jnp.float32),
            pltpu.VMEM((_BLOCK,), jnp.int32),
            pltpu.VMEM((_BLOCK,), jnp.int32),
            pltpu.VMEM((_BLOCK,), jnp.int32),
            pltpu.VMEM((_BLOCK,), jnp.int32),
            pltpu.VMEM((_BLOCK, d), jnp.float32),
            pltpu.VMEM((_BLOCK, d), jnp.float32),
            pltpu.VMEM((_BLOCK, d), jnp.float32),
            pltpu.VMEM((_BLOCK, d), jnp.float32),
            pltpu.VMEM((_BLOCK, d), jnp.float32),
            pltpu.SemaphoreType.DMA,
            pltpu.SemaphoreType.DMA,
        ],
    )
    def sc_kernel(table_hbm, ts_hbm, out_hbm,
                  ts_v, frac_v, il0, ir0, il1, ir1,
                  l0, r0, l1, r1, out_v, sg0, sg1):
        il, ir, lv_, rv_ = (il0, il1), (ir0, ir1), (l0, l1), (r0, r1)
        sg = (sg0, sg1)
        wid = lax.axis_index("s") * _NUM_CORES + lax.axis_index("c")
        t0 = wid * rows_per_w
        pltpu.sync_copy(ts_hbm.at[pl.ds(t0, rows_per_w)], ts_v)

        def build_lists(b, slot):
            @pl.loop(0, _BLOCK, step=_LANES)
            def _(i):
                tv = ts_v[pl.ds(b * _BLOCK + i, _LANES)]
                idx = (tv * float(nc - 2)).astype(jnp.int32)
                frac_v[pl.ds(b * _BLOCK + i, _LANES)] = (
                    tv * float(nc - 1) - idx.astype(jnp.float32))
                il[slot][pl.ds(i, _LANES)] = idx
                ir[slot][pl.ds(i, _LANES)] = idx + 1

        half = _BLOCK // 2

        def launch(slot):
            for h in (0, half):
                pltpu.async_copy(table_hbm.at[il[slot].at[pl.ds(h, half)]],
                                 lv_[slot].at[pl.ds(h, half)], sg[slot])
                pltpu.async_copy(table_hbm.at[ir[slot].at[pl.ds(h, half)]],
                                 rv_[slot].at[pl.ds(h, half)], sg[slot])

        def wait(slot):
            for h in (0, half):
                pltpu.make_async_copy(
                    table_hbm.at[il[slot].at[pl.ds(h, half)]],
                    lv_[slot].at[pl.ds(h, half)], sg[slot]).wait()
                pltpu.make_async_copy(
                    table_hbm.at[ir[slot].at[pl.ds(h, half)]],
                    rv_[slot].at[pl.ds(h, half)], sg[slot]).wait()

        def lerp_and_store(b, slot):
            @pl.loop(0, _BLOCK)
            def _(r):
                fv = plsc.load_gather(
                    frac_v, [jnp.full((_LANES,), b * _BLOCK + r, jnp.int32)])
                omf = 1.0 - fv
                for c in range(0, d, _LANES):
                    lo = lv_[slot][r, pl.ds(c, _LANES)]
                    hi = rv_[slot][r, pl.ds(c, _LANES)]
                    out_v[r, pl.ds(c, _LANES)] = omf * lo + fv * hi

            pltpu.sync_copy(
                out_v, out_hbm.at[pl.ds(t0 + b * _BLOCK, _BLOCK)])

        build_lists(0, 0)
        launch(0)

        @pl.loop(0, nblk, step=2)
        def _(b):
            for off, slot in ((0, 0), (1, 1)):
                bb = b + off

                @pl.when(bb + 1 < nblk)
                def _():
                    build_lists(bb + 1, 1 - slot)
                    launch(1 - slot)

                wait(slot)
                lerp_and_store(bb, slot)

    return sc_kernel


def kernel(timestamps, control_points):
    n_start, nc, two = control_points.shape
    t_total = timestamps.shape[0]
    d = n_start * two
    table = control_points.transpose(1, 0, 2).reshape(nc, d)
    sc_kernel = _build_sc_lerp_gather(t_total, nc, d)
    out_rows = sc_kernel(table, timestamps)
    return out_rows.reshape(t_total, n_start, two).transpose(1, 0, 2)


# speedup vs baseline: 1.0029x; 1.0029x over previous
import dataclasses
import functools

import jax
import jax.numpy as jnp
from jax import lax
from jax.experimental import pallas as pl
from jax.experimental.pallas import tpu as pltpu
from jax.experimental.pallas import tpu_sc as plsc

_NUM_CORES = 2
_NUM_SUBCORES = 16
_NW = _NUM_CORES * _NUM_SUBCORES
_LANES = 16
_BLOCK = 128


@functools.lru_cache(maxsize=None)
def _build_sc_lerp_gather(t_total: int, nc: int, d: int):
    assert t_total % (_NW * _BLOCK) == 0
    rows_per_w = t_total // _NW
    nblk = rows_per_w // _BLOCK
    assert nblk % 2 == 0
    mesh = plsc.VectorSubcoreMesh(core_axis_name="c", subcore_axis_name="s")
    cparams = pltpu.CompilerParams()
    if "needs_layout_passes" in pltpu.CompilerParams.__dataclass_fields__:
        cparams = dataclasses.replace(cparams, needs_layout_passes=False)

    @functools.partial(
        pl.kernel,
        out_type=jax.ShapeDtypeStruct((t_total, d), jnp.float32),
        mesh=mesh,
        compiler_params=cparams,
        scratch_types=[
            pltpu.VMEM((rows_per_w,), jnp.float32),
            pltpu.VMEM((rows_per_w,), jnp.float32),
            pltpu.VMEM((_BLOCK,), jnp.int32),
            pltpu.VMEM((_BLOCK,), jnp.int32),
            pltpu.VMEM((_BLOCK,), jnp.int32),
            pltpu.VMEM((_BLOCK,), jnp.int32),
            pltpu.VMEM((_BLOCK, d), jnp.float32),
            pltpu.VMEM((_BLOCK, d), jnp.float32),
            pltpu.VMEM((_BLOCK, d), jnp.float32),
            pltpu.VMEM((_BLOCK, d), jnp.float32),
            pltpu.VMEM((_BLOCK, d), jnp.float32),
            pltpu.SemaphoreType.DMA,
            pltpu.SemaphoreType.DMA,
        ],
    )
    def sc_kernel(table_hbm, ts_hbm, out_hbm,
                  ts_v, frac_v, il0, ir0, il1, ir1,
                  l0, r0, l1, r1, out_v, sg0, sg1):
        il, ir, lv_, rv_ = (il0, il1), (ir0, ir1), (l0, l1), (r0, r1)
        sg = (sg0, sg1)
        wid = lax.axis_index("s") * _NUM_CORES + lax.axis_index("c")
        t0 = wid * rows_per_w
        pltpu.sync_copy(ts_hbm.at[pl.ds(t0, rows_per_w)], ts_v)

        def build_lists(b, slot):
            @pl.loop(0, _BLOCK, step=_LANES)
            def _(i):
                tv = ts_v[pl.ds(b * _BLOCK + i, _LANES)]
                idx = (tv * float(nc - 2)).astype(jnp.int32)
                frac_v[pl.ds(b * _BLOCK + i, _LANES)] = (
                    tv * float(nc - 1) - idx.astype(jnp.float32))
                il[slot][pl.ds(i, _LANES)] = idx
                ir[slot][pl.ds(i, _LANES)] = idx + 1

        def launch(slot):
            pltpu.async_copy(table_hbm.at[il[slot]], lv_[slot], sg[slot])
            pltpu.async_copy(table_hbm.at[ir[slot]], rv_[slot], sg[slot])

        def wait(slot):
            pltpu.make_async_copy(table_hbm.at[il[slot]], lv_[slot],
                                  sg[slot]).wait()
            pltpu.make_async_copy(table_hbm.at[ir[slot]], rv_[slot],
                                  sg[slot]).wait()

        def lerp_and_store(b, slot):
            @pl.loop(0, _BLOCK)
            def _(r):
                fv = plsc.load_gather(
                    frac_v, [jnp.full((_LANES,), b * _BLOCK + r, jnp.int32)])
                omf = 1.0 - fv
                for c in range(0, d, _LANES):
                    lo = lv_[slot][r, pl.ds(c, _LANES)]
                    hi = rv_[slot][r, pl.ds(c, _LANES)]
                    out_v[r, pl.ds(c, _LANES)] = omf * lo + fv * hi

            pltpu.sync_copy(
                out_v, out_hbm.at[pl.ds(t0 + b * _BLOCK, _BLOCK)])

        build_lists(0, 0)
        launch(0)

        @pl.loop(0, nblk, step=2)
        def _(b):
            for off, slot in ((0, 0), (1, 1)):
                bb = b + off

                @pl.when(bb + 1 < nblk)
                def _():
                    build_lists(bb + 1, 1 - slot)
                    launch(1 - slot)

                wait(slot)
                lerp_and_store(bb, slot)

    return sc_kernel


def kernel(timestamps, control_points):
    n_start, nc, two = control_points.shape
    t_total = timestamps.shape[0]
    d = n_start * two
    table = control_points.transpose(1, 0, 2).reshape(nc, d)
    sc_kernel = _build_sc_lerp_gather(t_total, nc, d)
    out_rows = sc_kernel(table, timestamps)
    return out_rows.reshape(t_total, n_start, two).transpose(1, 0, 2)
